# baseline (speedup 1.0000x reference)
import jax
import jax.numpy as jnp
from jax import lax
from jax.experimental import pallas as pl
from jax.experimental.pallas import tpu as pltpu

_B, _T, _D, _E = 4, 8192, 1024, 64
_CHR = 512
_NCH = (_B * _T) // _CHR
_CPB = _T // _CHR
_NBUF = 8
_AW = 32


def _body(x_hbm, w_ref, bias_ref, idx_ref, probs_ref,
          b0, b1, b2, b3, b4, b5, b6, b7, acc_ref,
          s0, s1, s2, s3, s4, s5, s6, s7):
    bufs = (b0, b1, b2, b3, b4, b5, b6, b7)
    sems = (s0, s1, s2, s3, s4, s5, s6, s7)

    def dma(c, k):
        return pltpu.make_async_copy(
            x_hbm.at[pl.ds(c * _CHR, _CHR)], bufs[k], sems[k])

    for k in range(_NBUF):
        dma(k, k).start()

    for c in range(_NCH):
        k = c % _NBUF
        dma(c, k).wait()
        buf = bufs[k]
        b = c // _CPB
        s = buf[0:_AW]
        for i in range(1, _CHR // _AW):
            s = s + buf[i * _AW:(i + 1) * _AW]
        if c % _CPB == 0:
            acc_ref[b] = s
        else:
            acc_ref[b] += s
        if c + _NBUF < _NCH:
            dma(c + _NBUF, k).start()

    pooled = jnp.sum(acc_ref[...], axis=1) * (1.0 / _T)
    logits = lax.dot_general(
        pooled, w_ref[...], (((1,), (1,)), ((), ())),
        preferred_element_type=jnp.float32) + bias_ref[...][None, :]
    m = jnp.max(logits, axis=-1, keepdims=True)
    e = jnp.exp(logits - m)
    probs = e / jnp.sum(e, axis=-1, keepdims=True)
    probs_ref[...] = probs
    idx_ref[...] = jnp.argmax(probs, axis=-1).astype(jnp.int32)


def kernel(x, W, expert_bias):
    xf = x.reshape(_B * _T, _D)
    idx, probs = pl.pallas_call(
        _body,
        in_specs=[
            pl.BlockSpec(memory_space=pl.ANY),
            pl.BlockSpec((_E, _D), lambda: (0, 0)),
            pl.BlockSpec((_E,), lambda: (0,)),
        ],
        out_shape=(jax.ShapeDtypeStruct((_B,), jnp.int32),
                   jax.ShapeDtypeStruct((_B, _E), jnp.float32)),
        scratch_shapes=(
            [pltpu.VMEM((_CHR, _D), jnp.float32) for _ in range(_NBUF)]
            + [pltpu.VMEM((_B, _AW, _D), jnp.float32)]
            + [pltpu.SemaphoreType.DMA for _ in range(_NBUF)]
        ),
    )(xf, W, expert_bias)
    return idx, probs
